# Initial kernel scaffold; baseline (speedup 1.0000x reference)
import jax
import jax.numpy as jnp
from jax.experimental import pallas as pl


def kernel(h, edge_index, W, b, fc_W, fc_b):
    raise NotImplementedError("write your pallas kernel here")


# speedup vs baseline: 21.7885x; 21.7885x over previous
import functools

import jax
import jax.numpy as jnp
from jax import lax
from jax.experimental import pallas as pl
from jax.experimental.pallas import tpu as pltpu
from jax.experimental.pallas import tpu_sc as plsc

N = 10000
E = 320000
D = 128

NT = 16
EPT = E // NT
NBINS = 16384
COLS = NBINS // NT
N_PAD = 10 * COLS
VPC = COLS // 16


def _nrsqrt(x):
    i = plsc.bitcast(x, jnp.int32)
    y = plsc.bitcast(jnp.int32(0x5F3759DF) - (i >> 1), jnp.float32)
    for _ in range(3):
        y = y * (1.5 - 0.5 * x * y * y)
    return y


def _sc_weights(src, dst):
    mesh = plsc.VectorSubcoreMesh(
        core_axis_name="c", subcore_axis_name="s", num_cores=1,
        num_subcores=NT)

    @functools.partial(
        pl.kernel,
        out_type=jax.ShapeDtypeStruct((N_PAD,), jnp.float32),
        mesh=mesh,
        compiler_params=pltpu.CompilerParams(needs_layout_passes=False),
        scratch_types=[
            pltpu.VMEM((EPT,), jnp.int32),
            pltpu.VMEM((EPT,), jnp.int32),
            pltpu.VMEM((NBINS,), jnp.float32),
            pltpu.VMEM((NBINS,), jnp.float32),
            pltpu.VMEM((NBINS,), jnp.float32),
            pltpu.VMEM((COLS,), jnp.float32),
            pltpu.VMEM((COLS,), jnp.float32),
            pltpu.VMEM((COLS,), jnp.float32),
            pltpu.VMEM((COLS,), jnp.float32),
            pltpu.VMEM_SHARED((NT, NBINS), jnp.float32),
            pltpu.VMEM_SHARED((NT, NBINS), jnp.float32),
            pltpu.VMEM_SHARED((NBINS,), jnp.float32),
        ],
    )
    def kern(src_hbm, dst_hbm, w_hbm, sv, dv, ha, hb, rsqf,
             t1, buf_a, buf_b, rsqo, stA, stB, shR):
        sid = lax.axis_index("s")
        n0 = sid * COLS
        zeros16 = jnp.zeros((16,), jnp.float32)
        ones16 = jnp.ones((16,), jnp.float32)

        def zero_hist(h):
            def zb(k, _):
                h[pl.ds(k * 16, 16)] = zeros16
                return _
            lax.fori_loop(0, NBINS // 16, zb, None)

        def reduce_cols(st, out):
            for r in range(NT):
                pltpu.sync_copy(st.at[r, pl.ds(n0, COLS)], t1)

                def acc(k, _):
                    s = pl.ds(k * 16, 16)
                    if r == 0:
                        out[s] = t1[s]
                    else:
                        out[s] = out[s] + t1[s]
                    return _
                lax.fori_loop(0, VPC, acc, None)

        e0 = sid * EPT
        pltpu.sync_copy(src_hbm.at[pl.ds(e0, EPT)], sv)
        pltpu.sync_copy(dst_hbm.at[pl.ds(e0, EPT)], dv)
        zero_hist(ha)
        zero_hist(hb)

        def hist_body(i, _):
            o = i * 16
            plsc.addupdate_scatter(ha, [sv[pl.ds(o, 16)]], ones16)
            plsc.addupdate_scatter(hb, [dv[pl.ds(o, 16)]], ones16)
            return _
        lax.fori_loop(0, EPT // 16, hist_body, None)
        pltpu.sync_copy(ha, stA.at[sid])
        pltpu.sync_copy(hb, stB.at[sid])
        plsc.subcore_barrier()

        reduce_cols(stA, buf_a)
        reduce_cols(stB, buf_b)

        def rsq_body(k, _):
            s = pl.ds(k * 16, 16)
            rsqo[s] = _nrsqrt(jnp.maximum(buf_a[s], 1.0))
            buf_b[s] = _nrsqrt(jnp.maximum(buf_b[s], 1.0))
            return _
        lax.fori_loop(0, VPC, rsq_body, None)
        pltpu.sync_copy(buf_b, shR.at[pl.ds(n0, COLS)])
        plsc.subcore_barrier()
        pltpu.sync_copy(shR, rsqf)

        zero_hist(hb)

        def edge_body(i, _):
            o = i * 16
            v = plsc.load_gather(rsqf, [dv[pl.ds(o, 16)]])
            plsc.addupdate_scatter(hb, [sv[pl.ds(o, 16)]], v)
            return _
        lax.fori_loop(0, EPT // 16, edge_body, None)
        pltpu.sync_copy(hb, stB.at[sid])
        plsc.subcore_barrier()

        reduce_cols(stB, buf_a)

        def mul_body(k, _):
            s = pl.ds(k * 16, 16)
            buf_a[s] = buf_a[s] * rsqo[s]
            return _
        lax.fori_loop(0, VPC, mul_body, None)

        @pl.when(sid < N_PAD // COLS)
        def _():
            pltpu.sync_copy(buf_a, w_hbm.at[pl.ds(n0, COLS)])

    return kern(src, dst)


def _tc_finish(w2, h_pad, W, b2, fc_W, fcb2):
    def body(w_ref, h_ref, W_ref, b_ref, fcW_ref, fcb_ref, o_ref):
        v = lax.dot_general(w_ref[...], h_ref[...],
                            (((1,), (0,)), ((), ())),
                            preferred_element_type=jnp.float32,
                            precision=lax.Precision.HIGHEST)
        hg = lax.dot_general(v, W_ref[...],
                             (((1,), (0,)), ((), ())),
                             preferred_element_type=jnp.float32,
                             precision=lax.Precision.HIGHEST)
        hg = hg * (1.0 / N) + b_ref[...]
        out = lax.dot_general(hg, fcW_ref[...],
                              (((1,), (1,)), ((), ())),
                              preferred_element_type=jnp.float32,
                              precision=lax.Precision.HIGHEST)
        o_ref[...] = out + fcb_ref[...]

    return pl.pallas_call(
        body,
        out_shape=jax.ShapeDtypeStruct((1, D), jnp.float32),
    )(w2, h_pad, W, b2, fc_W, fcb2)


def kernel(h, edge_index, W, b, fc_W, fc_b):
    src = edge_index[0]
    dst = edge_index[1]
    w = _sc_weights(src, dst)
    h_pad = jnp.pad(h, ((0, N_PAD - N), (0, 0)))
    return _tc_finish(w.reshape(1, N_PAD), h_pad, W,
                      b.reshape(1, D), fc_W, fc_b.reshape(1, D))


# speedup vs baseline: 43.2777x; 1.9863x over previous
import functools

import jax
import jax.numpy as jnp
from jax import lax
from jax.experimental import pallas as pl
from jax.experimental.pallas import tpu as pltpu
from jax.experimental.pallas import tpu_sc as plsc

N = 10000
E = 320000
D = 128

NT = 16
EPT = E // NT
NBINS = 10240
COLS = NBINS // NT
VPC = COLS // 16


def _nrsqrt(x):
    i = plsc.bitcast(x, jnp.int32)
    y = plsc.bitcast(jnp.int32(0x5F3759DF) - (i >> 1), jnp.float32)
    for _ in range(3):
        y = y * (1.5 - 0.5 * x * y * y)
    return y


def _sc_edge_work(src, dst):
    mesh = plsc.VectorSubcoreMesh(
        core_axis_name="c", subcore_axis_name="s", num_cores=1,
        num_subcores=NT)

    @functools.partial(
        pl.kernel,
        out_type=(
            jax.ShapeDtypeStruct((NT, NBINS), jnp.float32),
            jax.ShapeDtypeStruct((1, NBINS), jnp.float32),
        ),
        mesh=mesh,
        compiler_params=pltpu.CompilerParams(needs_layout_passes=False),
        scratch_types=[
            pltpu.VMEM((EPT,), jnp.int32),
            pltpu.VMEM((EPT,), jnp.int32),
            pltpu.VMEM((NBINS,), jnp.float32),
            pltpu.VMEM((NBINS,), jnp.float32),
            pltpu.VMEM((NBINS,), jnp.float32),
            pltpu.VMEM((NT, COLS), jnp.float32),
            pltpu.VMEM((COLS,), jnp.float32),
            pltpu.VMEM_SHARED((NT, NBINS), jnp.float32),
            pltpu.VMEM_SHARED((NT, NBINS), jnp.float32),
            pltpu.VMEM_SHARED((NBINS,), jnp.float32),
            pltpu.SemaphoreType.DMA,
            pltpu.SemaphoreType.DMA,
        ],
    )
    def kern(src_hbm, dst_hbm, sp_hbm, ro_hbm, sv, dv, ha, hb, rsqf,
             t16, buf, stA, stB, shR, sem1, sem2):
        sid = lax.axis_index("s")
        n0 = sid * COLS
        zeros16 = jnp.zeros((16,), jnp.float32)
        ones16 = jnp.ones((16,), jnp.float32)

        e0 = sid * EPT
        cps = pltpu.async_copy(src_hbm.at[pl.ds(e0, EPT)], sv, sem1)
        cpd = pltpu.async_copy(dst_hbm.at[pl.ds(e0, EPT)], dv, sem2)

        @plsc.parallel_loop(0, NBINS // 16, unroll=16)
        def _(k):
            ha[pl.ds(k * 16, 16)] = zeros16
            hb[pl.ds(k * 16, 16)] = zeros16

        cps.wait()
        cpd.wait()

        @plsc.parallel_loop(0, EPT // 16, unroll=8)
        def _(i):
            o = i * 16
            plsc.addupdate_scatter(ha, [sv[pl.ds(o, 16)]], ones16)
            plsc.addupdate_scatter(hb, [dv[pl.ds(o, 16)]], ones16)

        pltpu.sync_copy(ha, stA.at[sid])
        pltpu.sync_copy(hb, stB.at[sid])
        plsc.subcore_barrier()

        pltpu.sync_copy(stA.at[:, pl.ds(n0, COLS)], t16)

        @plsc.parallel_loop(0, VPC, unroll=4)
        def _(k):
            s_ = pl.ds(k * 16, 16)
            acc = t16[0, s_]
            for r in range(1, NT):
                acc = acc + t16[r, s_]
            buf[s_] = _nrsqrt(jnp.maximum(acc, 1.0))

        pltpu.sync_copy(buf, ro_hbm.at[0, pl.ds(n0, COLS)])

        pltpu.sync_copy(stB.at[:, pl.ds(n0, COLS)], t16)

        @plsc.parallel_loop(0, VPC, unroll=4)
        def _(k):
            s_ = pl.ds(k * 16, 16)
            acc = t16[0, s_]
            for r in range(1, NT):
                acc = acc + t16[r, s_]
            buf[s_] = _nrsqrt(jnp.maximum(acc, 1.0))

        pltpu.sync_copy(buf, shR.at[pl.ds(n0, COLS)])
        plsc.subcore_barrier()
        pltpu.sync_copy(shR, rsqf)

        @plsc.parallel_loop(0, NBINS // 16, unroll=16)
        def _(k):
            hb[pl.ds(k * 16, 16)] = zeros16

        @plsc.parallel_loop(0, EPT // 16, unroll=8)
        def _(i):
            o = i * 16
            v = plsc.load_gather(rsqf, [dv[pl.ds(o, 16)]])
            plsc.addupdate_scatter(hb, [sv[pl.ds(o, 16)]], v)

        pltpu.sync_copy(hb, sp_hbm.at[sid])

    return kern(src, dst)


def _tc_finish(s_parts, rsqo, h_pad, W, b2, fc_W, fcb2):
    def body(sp_ref, ro_ref, h_ref, W_ref, b_ref, fcW_ref, fcb_ref, o_ref):
        s = jnp.sum(sp_ref[...], axis=0, keepdims=True)
        w2 = s * ro_ref[...]
        v = lax.dot_general(w2, h_ref[...],
                            (((1,), (0,)), ((), ())),
                            preferred_element_type=jnp.float32,
                            precision=lax.Precision.HIGHEST)
        hg = lax.dot_general(v, W_ref[...],
                             (((1,), (0,)), ((), ())),
                             preferred_element_type=jnp.float32,
                             precision=lax.Precision.HIGHEST)
        hg = hg * (1.0 / N) + b_ref[...]
        out = lax.dot_general(hg, fcW_ref[...],
                              (((1,), (1,)), ((), ())),
                              preferred_element_type=jnp.float32,
                              precision=lax.Precision.HIGHEST)
        o_ref[...] = out + fcb_ref[...]

    return pl.pallas_call(
        body,
        out_shape=jax.ShapeDtypeStruct((1, D), jnp.float32),
    )(s_parts, rsqo, h_pad, W, b2, fc_W, fcb2)


def kernel(h, edge_index, W, b, fc_W, fc_b):
    src = edge_index[0]
    dst = edge_index[1]
    s_parts, rsqo = _sc_edge_work(src, dst)
    h_pad = jnp.pad(h, ((0, NBINS - N), (0, 0)))
    return _tc_finish(s_parts, rsqo, h_pad, W,
                      b.reshape(1, D), fc_W, fc_b.reshape(1, D))


# speedup vs baseline: 55.1956x; 1.2754x over previous
import functools

import jax
import jax.numpy as jnp
from jax import lax
from jax.experimental import pallas as pl
from jax.experimental.pallas import tpu as pltpu
from jax.experimental.pallas import tpu_sc as plsc

N = 10000
E = 320000
D = 128

NT = 16
EPT = E // NT
NBINS = 10240
COLS = NBINS // NT
VPC = COLS // 16


def _nrsqrt(x):
    i = plsc.bitcast(x, jnp.int32)
    y = plsc.bitcast(jnp.int32(0x5F3759DF) - (i >> 1), jnp.float32)
    for _ in range(3):
        y = y * (1.5 - 0.5 * x * y * y)
    return y


def _sc_edge_work(edge_index):
    mesh = plsc.VectorSubcoreMesh(
        core_axis_name="c", subcore_axis_name="s", num_cores=1,
        num_subcores=NT)

    @functools.partial(
        pl.kernel,
        out_type=(
            jax.ShapeDtypeStruct((NT, NBINS), jnp.float32),
            jax.ShapeDtypeStruct((1, NBINS), jnp.float32),
        ),
        mesh=mesh,
        compiler_params=pltpu.CompilerParams(needs_layout_passes=False,
                                             skip_device_barrier=True),
        scratch_types=[
            pltpu.VMEM((EPT,), jnp.int32),
            pltpu.VMEM((EPT,), jnp.int32),
            pltpu.VMEM((NBINS,), jnp.float32),
            pltpu.VMEM((NBINS,), jnp.float32),
            pltpu.VMEM((NBINS,), jnp.float32),
            pltpu.VMEM((NT, COLS), jnp.float32),
            pltpu.VMEM((NT, COLS), jnp.float32),
            pltpu.VMEM((COLS,), jnp.float32),
            pltpu.VMEM((COLS,), jnp.float32),
            pltpu.VMEM_SHARED((NT, NBINS), jnp.float32),
            pltpu.VMEM_SHARED((NT, NBINS), jnp.float32),
            pltpu.VMEM_SHARED((NBINS,), jnp.float32),
            pltpu.SemaphoreType.DMA,
            pltpu.SemaphoreType.DMA,
        ],
    )
    def kern(ei_hbm, sp_hbm, ro_hbm, sv, dv, ha, hb, rsqf,
             t16, t16b, buf, buf2, stA, stB, shR, sem1, sem2):
        sid = lax.axis_index("s")
        n0 = sid * COLS
        zeros16 = jnp.zeros((16,), jnp.float32)
        ones16 = jnp.ones((16,), jnp.float32)

        e0 = sid * EPT
        cps = pltpu.async_copy(ei_hbm.at[pl.ds(e0, EPT)], sv, sem1)
        cpd = pltpu.async_copy(ei_hbm.at[pl.ds(E + e0, EPT)], dv, sem2)

        @plsc.parallel_loop(0, NBINS // 16, unroll=16)
        def _(k):
            ha[pl.ds(k * 16, 16)] = zeros16
            hb[pl.ds(k * 16, 16)] = zeros16

        cps.wait()
        cpd.wait()

        @plsc.parallel_loop(0, EPT // 16, unroll=8)
        def _(i):
            o = i * 16
            plsc.addupdate_scatter(ha, [sv[pl.ds(o, 16)]], ones16)
            plsc.addupdate_scatter(hb, [dv[pl.ds(o, 16)]], ones16)

        cpa = pltpu.async_copy(ha, stA.at[sid], sem1)
        cpb = pltpu.async_copy(hb, stB.at[sid], sem2)
        cpa.wait()
        cpb.wait()
        plsc.subcore_barrier()

        cpb2 = pltpu.async_copy(stB.at[:, pl.ds(n0, COLS)], t16b, sem2)
        pltpu.sync_copy(stA.at[:, pl.ds(n0, COLS)], t16)

        @plsc.parallel_loop(0, VPC, unroll=4)
        def _(k):
            s_ = pl.ds(k * 16, 16)
            acc = t16[0, s_]
            for r in range(1, NT):
                acc = acc + t16[r, s_]
            buf[s_] = _nrsqrt(jnp.maximum(acc, 1.0))

        cpo = pltpu.async_copy(buf, ro_hbm.at[0, pl.ds(n0, COLS)], sem1)
        cpb2.wait()

        @plsc.parallel_loop(0, VPC, unroll=4)
        def _(k):
            s_ = pl.ds(k * 16, 16)
            acc = t16b[0, s_]
            for r in range(1, NT):
                acc = acc + t16b[r, s_]
            buf2[s_] = _nrsqrt(jnp.maximum(acc, 1.0))

        pltpu.sync_copy(buf2, shR.at[pl.ds(n0, COLS)])
        cpo.wait()
        plsc.subcore_barrier()
        cpr = pltpu.async_copy(shR, rsqf, sem1)

        @plsc.parallel_loop(0, NBINS // 16, unroll=16)
        def _(k):
            hb[pl.ds(k * 16, 16)] = zeros16

        cpr.wait()

        @plsc.parallel_loop(0, EPT // 16, unroll=8)
        def _(i):
            o = i * 16
            v = plsc.load_gather(rsqf, [dv[pl.ds(o, 16)]])
            plsc.addupdate_scatter(hb, [sv[pl.ds(o, 16)]], v)

        pltpu.sync_copy(hb, sp_hbm.at[sid])

    return kern(edge_index.reshape(2 * E))


def _tc_finish(s_parts, rsqo, h, W, b2, fc_W, fcb2):
    def body(sp_ref, ro_ref, h_ref, W_ref, b_ref, fcW_ref, fcb_ref, o_ref):
        s = jnp.sum(sp_ref[...], axis=0, keepdims=True)
        w2 = (s * ro_ref[...])[:, :N]
        v = lax.dot_general(w2, h_ref[...],
                            (((1,), (0,)), ((), ())),
                            preferred_element_type=jnp.float32,
                            precision=lax.Precision.HIGHEST)
        hg = lax.dot_general(v, W_ref[...],
                             (((1,), (0,)), ((), ())),
                             preferred_element_type=jnp.float32,
                             precision=lax.Precision.HIGHEST)
        hg = hg * (1.0 / N) + b_ref[...]
        out = lax.dot_general(hg, fcW_ref[...],
                              (((1,), (1,)), ((), ())),
                              preferred_element_type=jnp.float32,
                              precision=lax.Precision.HIGHEST)
        o_ref[...] = out + fcb_ref[...]

    return pl.pallas_call(
        body,
        out_shape=jax.ShapeDtypeStruct((1, D), jnp.float32),
        compiler_params=pltpu.CompilerParams(skip_device_barrier=True),
    )(s_parts, rsqo, h, W, b2, fc_W, fcb2)


def kernel(h, edge_index, W, b, fc_W, fc_b):
    s_parts, rsqo = _sc_edge_work(edge_index)
    return _tc_finish(s_parts, rsqo, h, W,
                      b.reshape(1, D), fc_W, fc_b.reshape(1, D))
